# baseline (speedup 1.0000x reference)
import functools

import jax
import jax.numpy as jnp
from jax import lax
from jax.experimental import pallas as pl
from jax.experimental.pallas import tpu as pltpu
from jax.experimental.pallas import tpu_sc as plsc


def _proj_body(x_ref, w_ref, b_ref, o_ref):
    blk = x_ref.shape[0]
    x = x_ref[...]
    w = w_ref[...]
    ones = jnp.ones((32, 128), jnp.float32)
    s = jax.lax.dot_general(x * w, ones, (((1,), (0,)), ((), ())),
                            preferred_element_type=jnp.float32)
    r = jax.lax.broadcasted_iota(jnp.int32, (blk, 128), 0)
    l = jax.lax.broadcasted_iota(jnp.int32, (blk, 128), 1)
    m = jnp.where((r & 127) == l, s, 0.0)
    z = jnp.sum(m.reshape(blk // 128, 128, 128), axis=1)
    o_ref[...] = z.reshape(blk) + b_ref[0, 0]


def _project_table(table, W, b, blk=40960):
    n = table.shape[0]
    grid = (n + blk - 1) // blk
    return pl.pallas_call(
        _proj_body,
        grid=(grid,),
        in_specs=[
            pl.BlockSpec((blk, table.shape[1]), lambda i: (i, 0)),
            pl.BlockSpec((1, table.shape[1]), lambda i: (0, 0)),
            pl.BlockSpec((1, 1), lambda i: (0, 0)),
        ],
        out_specs=pl.BlockSpec((blk,), lambda i: (i,)),
        out_shape=jax.ShapeDtypeStruct((n,), jnp.float32),
        compiler_params=pltpu.CompilerParams(
            dimension_semantics=("parallel",)),
    )(table, W.reshape(1, -1), b.reshape(1, 1))


def _make_gather(ntot, ch):
    info = plsc.get_sparse_core_info()
    nc, ns = info.num_cores, info.num_subcores
    nw = nc * ns
    per_w = ntot // nw
    n_ch = per_w // ch
    mesh = plsc.VectorSubcoreMesh(core_axis_name="c", subcore_axis_name="s")

    @functools.partial(
        pl.kernel,
        mesh=mesh,
        out_type=jax.ShapeDtypeStruct((ntot,), jnp.float32),
        scratch_types=[
            pltpu.VMEM((ch,), jnp.int32),
            pltpu.VMEM((ch,), jnp.int32),
            pltpu.VMEM((ch,), jnp.float32),
            pltpu.VMEM((ch,), jnp.float32),
            pltpu.SemaphoreType.DMA((2,)),
            pltpu.SemaphoreType.DMA((2,)),
            pltpu.SemaphoreType.DMA((2,)),
        ],
    )
    def gather_k(tw_hbm, qf_hbm, out_hbm, idx0, idx1, val0, val1,
                 isem, gsem, ssem):
        idx_v = (idx0, idx1)
        val_v = (val0, val1)
        wid = lax.axis_index("s") * nc + lax.axis_index("c")
        base = wid * per_w

        def idx_load(k):
            return pltpu.async_copy(
                qf_hbm.at[pl.ds(base + k * ch, ch)], idx_v[k % 2],
                isem.at[k % 2])

        loads = [idx_load(0)]
        stores = [None, None]
        for k in range(n_ch):
            if k + 1 < n_ch:
                loads.append(idx_load(k + 1))
            loads[k].wait()
            if stores[k % 2] is not None:
                stores[k % 2].wait()
            pltpu.async_copy(
                tw_hbm.at[idx_v[k % 2]], val_v[k % 2],
                gsem.at[k % 2]).wait()
            stores[k % 2] = pltpu.async_copy(
                val_v[k % 2], out_hbm.at[pl.ds(base + k * ch, ch)],
                ssem.at[k % 2])
        for st in stores:
            if st is not None:
                st.wait()

    return gather_k


def _proj2_body(x0_ref, x1_ref, w_ref, b_ref, o0_ref, o1_ref):
    for x_ref, o_ref in ((x0_ref, o0_ref), (x1_ref, o1_ref)):
        blk = x_ref.shape[0]
        x = x_ref[...]
        w = w_ref[...]
        ones = jnp.ones((32, 128), jnp.float32)
        s = jax.lax.dot_general(x * w, ones, (((1,), (0,)), ((), ())),
                                preferred_element_type=jnp.float32)
        r = jax.lax.broadcasted_iota(jnp.int32, (blk, 128), 0)
        l = jax.lax.broadcasted_iota(jnp.int32, (blk, 128), 1)
        m = jnp.where((r & 127) == l, s, 0.0)
        z = jnp.sum(m.reshape(blk // 128, 128, 128), axis=1)
        o_ref[...] = z.reshape(blk) + b_ref[0, 0]


def _project_table2(table, W, b, blk=16384):
    n = table.shape[0]
    nblk = (n + blk - 1) // blk
    grid = (nblk + 1) // 2
    half = grid * blk
    o0, o1 = pl.pallas_call(
        _proj2_body,
        grid=(grid,),
        in_specs=[
            pl.BlockSpec((blk, table.shape[1]), lambda i: (2 * i, 0)),
            pl.BlockSpec((blk, table.shape[1]), lambda i: (2 * i + 1, 0)),
            pl.BlockSpec((1, table.shape[1]), lambda i: (0, 0)),
            pl.BlockSpec((1, 1), lambda i: (0, 0)),
        ],
        out_specs=[
            pl.BlockSpec((blk,), lambda i: (i,)),
            pl.BlockSpec((blk,), lambda i: (i,)),
        ],
        out_shape=[
            jax.ShapeDtypeStruct((half,), jnp.float32),
            jax.ShapeDtypeStruct((half,), jnp.float32),
        ],
        compiler_params=pltpu.CompilerParams(
            dimension_semantics=("parallel",)),
    )(table, table, W.reshape(1, -1), b.reshape(1, 1))
    tw = jnp.stack(
        [o0.reshape(grid, blk), o1.reshape(grid, blk)], axis=1).reshape(-1)
    return tw[:n]


def kernel(q, table, W, b):
    tw = _project_table2(table, W, b)
    qf = q.reshape(-1)
    gather_k = _make_gather(qf.shape[0], ch=25600)
    yf = gather_k(tw, qf)
    return yf.reshape(q.shape)


# speedup vs baseline: 1.0093x; 1.0093x over previous
import functools

import jax
import jax.numpy as jnp
from jax import lax
from jax.experimental import pallas as pl
from jax.experimental.pallas import tpu as pltpu
from jax.experimental.pallas import tpu_sc as plsc


def _proj_body(x_ref, w_ref, b_ref, o_ref):
    blk = x_ref.shape[0]
    x = x_ref[...]
    w = w_ref[...]
    ones = jnp.ones((32, 128), jnp.float32)
    s = jax.lax.dot_general(x * w, ones, (((1,), (0,)), ((), ())),
                            preferred_element_type=jnp.float32)
    r = jax.lax.broadcasted_iota(jnp.int32, (blk, 128), 0)
    l = jax.lax.broadcasted_iota(jnp.int32, (blk, 128), 1)
    m = jnp.where((r & 127) == l, s, 0.0)
    z = jnp.sum(m.reshape(blk // 128, 128, 128), axis=1)
    o_ref[...] = z.reshape(blk) + b_ref[0, 0]


def _project_table(table, W, b, blk=40960):
    n = table.shape[0]
    grid = (n + blk - 1) // blk
    return pl.pallas_call(
        _proj_body,
        grid=(grid,),
        in_specs=[
            pl.BlockSpec((blk, table.shape[1]), lambda i: (i, 0)),
            pl.BlockSpec((1, table.shape[1]), lambda i: (0, 0)),
            pl.BlockSpec((1, 1), lambda i: (0, 0)),
        ],
        out_specs=pl.BlockSpec((blk,), lambda i: (i,)),
        out_shape=jax.ShapeDtypeStruct((n,), jnp.float32),
        compiler_params=pltpu.CompilerParams(
            dimension_semantics=("parallel",)),
    )(table, W.reshape(1, -1), b.reshape(1, 1))


def _make_gather(ntot, ch):
    info = plsc.get_sparse_core_info()
    nc, ns = info.num_cores, info.num_subcores
    nw = nc * ns
    per_w = ntot // nw
    n_ch = per_w // ch
    mesh = plsc.VectorSubcoreMesh(core_axis_name="c", subcore_axis_name="s")

    @functools.partial(
        pl.kernel,
        mesh=mesh,
        out_type=jax.ShapeDtypeStruct((ntot,), jnp.float32),
        scratch_types=[
            pltpu.VMEM((ch,), jnp.int32),
            pltpu.VMEM((ch,), jnp.int32),
            pltpu.VMEM((ch,), jnp.float32),
            pltpu.VMEM((ch,), jnp.float32),
            pltpu.SemaphoreType.DMA((2,)),
            pltpu.SemaphoreType.DMA((2,)),
            pltpu.SemaphoreType.DMA((2,)),
        ],
    )
    def gather_k(tw_hbm, qf_hbm, out_hbm, idx0, idx1, val0, val1,
                 isem, gsem, ssem):
        idx_v = (idx0, idx1)
        val_v = (val0, val1)
        wid = lax.axis_index("s") * nc + lax.axis_index("c")
        base = wid * per_w

        def idx_load(k):
            return pltpu.async_copy(
                qf_hbm.at[pl.ds(base + k * ch, ch)], idx_v[k % 2],
                isem.at[k % 2])

        loads = [idx_load(0)]
        stores = [None, None]
        for k in range(n_ch):
            if k + 1 < n_ch:
                loads.append(idx_load(k + 1))
            loads[k].wait()
            if stores[k % 2] is not None:
                stores[k % 2].wait()
            pltpu.async_copy(
                tw_hbm.at[idx_v[k % 2]], val_v[k % 2],
                gsem.at[k % 2]).wait()
            stores[k % 2] = pltpu.async_copy(
                val_v[k % 2], out_hbm.at[pl.ds(base + k * ch, ch)],
                ssem.at[k % 2])
        for st in stores:
            if st is not None:
                st.wait()

    return gather_k


def kernel(q, table, W, b):
    tw = _project_table(table, W, b)
    qf = q.reshape(-1)
    gather_k = _make_gather(qf.shape[0], ch=25600)
    yf = gather_k(tw, qf)
    return yf.reshape(q.shape)


# speedup vs baseline: 1.1680x; 1.1573x over previous
import functools

import jax
import jax.numpy as jnp
from jax import lax
from jax.experimental import pallas as pl
from jax.experimental.pallas import tpu as pltpu
from jax.experimental.pallas import tpu_sc as plsc


def _proj_body(x_ref, w_ref, b_ref, o_ref):
    blk = x_ref.shape[0]
    x = x_ref[...]
    w = w_ref[...]
    ones = jnp.ones((32, 128), jnp.float32)
    s = jax.lax.dot_general(x * w, ones, (((1,), (0,)), ((), ())),
                            preferred_element_type=jnp.float32)
    r = jax.lax.broadcasted_iota(jnp.int32, (blk, 128), 0)
    l = jax.lax.broadcasted_iota(jnp.int32, (blk, 128), 1)
    m = jnp.where((r & 127) == l, s, 0.0)
    z = jnp.sum(m.reshape(blk // 128, 128, 128), axis=1)
    o_ref[...] = z.reshape(blk) + b_ref[0, 0]


def _project_table(table, W, b, blk=40960):
    n = table.shape[0]
    grid = (n + blk - 1) // blk
    return pl.pallas_call(
        _proj_body,
        grid=(grid,),
        in_specs=[
            pl.BlockSpec((blk, table.shape[1]), lambda i: (i, 0)),
            pl.BlockSpec((1, table.shape[1]), lambda i: (0, 0)),
            pl.BlockSpec((1, 1), lambda i: (0, 0)),
        ],
        out_specs=pl.BlockSpec((blk,), lambda i: (i,)),
        out_shape=jax.ShapeDtypeStruct((n,), jnp.float32),
        compiler_params=pltpu.CompilerParams(
            dimension_semantics=("parallel",)),
    )(table, W.reshape(1, -1), b.reshape(1, 1))


def _make_gather(ntot, ch):
    info = plsc.get_sparse_core_info()
    nc, ns = info.num_cores, info.num_subcores
    nw = nc * ns
    per_w = ntot // nw
    n_ch = per_w // ch
    mesh = plsc.VectorSubcoreMesh(core_axis_name="c", subcore_axis_name="s")

    @functools.partial(
        pl.kernel,
        mesh=mesh,
        out_type=jax.ShapeDtypeStruct((ntot,), jnp.float32),
        scratch_types=[
            pltpu.VMEM((ch,), jnp.int32),
            pltpu.VMEM((ch,), jnp.int32),
            pltpu.VMEM((ch,), jnp.float32),
            pltpu.VMEM((ch,), jnp.float32),
            pltpu.SemaphoreType.DMA((2,)),
            pltpu.SemaphoreType.DMA((2,)),
            pltpu.SemaphoreType.DMA((2,)),
        ],
    )
    def gather_k(tw_hbm, qf_hbm, out_hbm, idx0, idx1, val0, val1,
                 isem, gsem, ssem):
        idx_v = (idx0, idx1)
        val_v = (val0, val1)
        wid = lax.axis_index("s") * nc + lax.axis_index("c")
        base = wid * per_w

        def idx_load(k):
            return pltpu.async_copy(
                qf_hbm.at[pl.ds(base + k * ch, ch)], idx_v[k % 2],
                isem.at[k % 2])

        loads = [idx_load(0)]
        stores = [None, None]
        for k in range(n_ch):
            if k + 1 < n_ch:
                loads.append(idx_load(k + 1))
            loads[k].wait()
            if stores[k % 2] is not None:
                stores[k % 2].wait()
            pltpu.async_copy(
                tw_hbm.at[idx_v[k % 2]], val_v[k % 2],
                gsem.at[k % 2]).wait()
            stores[k % 2] = pltpu.async_copy(
                val_v[k % 2], out_hbm.at[pl.ds(base + k * ch, ch)],
                ssem.at[k % 2])
        for st in stores:
            if st is not None:
                st.wait()

    return gather_k


def _make_gather_spmem(ntot, nvoc, ch):
    info = plsc.get_sparse_core_info()
    nc, ns = info.num_cores, info.num_subcores
    nw = nc * ns
    per_w = ntot // nw
    n_ch = per_w // ch
    mesh = plsc.VectorSubcoreMesh(core_axis_name="c", subcore_axis_name="s")

    @functools.partial(
        pl.kernel,
        mesh=mesh,
        out_type=jax.ShapeDtypeStruct((ntot,), jnp.float32),
        scratch_types=[
            pltpu.VMEM_SHARED((nvoc,), jnp.float32),
            pltpu.VMEM((ch,), jnp.int32),
            pltpu.VMEM((ch,), jnp.int32),
            pltpu.VMEM((ch,), jnp.float32),
            pltpu.VMEM((ch,), jnp.float32),
            pltpu.SemaphoreType.DMA((2,)),
            pltpu.SemaphoreType.DMA((2,)),
            pltpu.SemaphoreType.DMA((2,)),
        ],
    )
    def gather_k(tw_hbm, qf_hbm, out_hbm, tw_sh, idx0, idx1, val0, val1,
                 isem, gsem, ssem):
        idx_v = (idx0, idx1)
        val_v = (val0, val1)
        sid = lax.axis_index("s")
        wid = sid * nc + lax.axis_index("c")
        base = wid * per_w

        @pl.when(sid == 0)
        def _():
            pltpu.sync_copy(tw_hbm, tw_sh)

        def idx_load(k):
            return pltpu.async_copy(
                qf_hbm.at[pl.ds(base + k * ch, ch)], idx_v[k % 2],
                isem.at[k % 2])

        loads = [idx_load(0), idx_load(1)]
        plsc.subcore_barrier()
        stores = [None, None]
        for k in range(n_ch):
            if k + 2 < n_ch:
                loads.append(idx_load(k + 2))
            loads[k].wait()
            if stores[k % 2] is not None:
                stores[k % 2].wait()
            pltpu.async_copy(
                tw_sh.at[idx_v[k % 2]], val_v[k % 2],
                gsem.at[k % 2]).wait()
            stores[k % 2] = pltpu.async_copy(
                val_v[k % 2], out_hbm.at[pl.ds(base + k * ch, ch)],
                ssem.at[k % 2])
        for st in stores:
            if st is not None:
                st.wait()

    return gather_k


def kernel(q, table, W, b):
    tw = _project_table(table, W, b)
    qf = q.reshape(-1)
    gather_k = _make_gather_spmem(qf.shape[0], tw.shape[0], ch=12800)
    yf = gather_k(tw, qf)
    return yf.reshape(q.shape)
